# Initial kernel scaffold; baseline (speedup 1.0000x reference)
import jax
import jax.numpy as jnp
from jax.experimental import pallas as pl


def kernel(x, edge_idx, W1, b1, g1, be1, W2, b2, g2, be2, W3, b3):
    raise NotImplementedError("write your pallas kernel here")


# speedup vs baseline: 7.4051x; 7.4051x over previous
import functools

import jax
import jax.numpy as jnp
from jax import lax
from jax.experimental import pallas as pl
from jax.experimental.pallas import tpu as pltpu
from jax.experimental.pallas import tpu_sc as plsc

N = 10000
E = 160000
F_IN = 256
H = 512
C = 40
EPS = 1e-5

NS = 16
NC = 2
ET = E // NS
BB = 128
NB = (ET + BB - 1) // BB
ETP = NB * BB
NPAD = ETP
RPT = NPAD // NS
MB = 1000
GR = N // MB

_mesh = plsc.VectorSubcoreMesh(core_axis_name="c", subcore_axis_name="s")


def _sds(shape, dtype=jnp.float32):
    return jax.ShapeDtypeStruct(shape, dtype)


def _deg_body(dst_hbm, ones_hbm, zeros_hbm, deg_hbm, idx_v, ones_v, acc_sh):
    c = lax.axis_index("c")
    s = lax.axis_index("s")
    pltpu.sync_copy(dst_hbm.at[s], idx_v)
    pltpu.sync_copy(ones_hbm, ones_v)
    pltpu.sync_copy(zeros_hbm.at[pl.ds(s * RPT, RPT)],
                    acc_sh.at[pl.ds(s * RPT, RPT)])
    plsc.subcore_barrier()

    def body(j, carry):
        jb = 2 * j + c

        @pl.when(jb < NB)
        def _():
            pltpu.sync_copy(ones_v, acc_sh.at[idx_v.at[jb]], add=True)
        return carry

    lax.fori_loop(0, (NB + 1) // 2, body, 0)
    plsc.subcore_barrier()
    pltpu.sync_copy(acc_sh.at[pl.ds(s * RPT, RPT)],
                    deg_hbm.at[c, pl.ds(s * RPT, RPT)])


_deg_call = pl.kernel(
    _deg_body,
    out_type=_sds((NC, NPAD, 128)),
    mesh=_mesh,
    scratch_types=[
        pltpu.VMEM((NB, BB), jnp.int32),
        pltpu.VMEM((BB, 128), jnp.float32),
        pltpu.VMEM_SHARED((NPAD, 128), jnp.float32),
    ],
)


def _agg_wide_body(u0, u1, u2, u3, src_hbm, dst_hbm, agg_hbm,
                   srcv, dstv, rowbuf, acc_sh):
    c = lax.axis_index("c")
    s = lax.axis_index("s")
    pltpu.sync_copy(src_hbm.at[s], srcv)
    pltpu.sync_copy(dst_hbm.at[s], dstv)
    u_refs = (u0, u1, u2, u3)
    for chunk in range(4):
        u_ref = u_refs[chunk]

        @pl.when(c == chunk // 2)
        def _():
            pltpu.sync_copy(u_ref.at[pl.ds(s * RPT, RPT)],
                            acc_sh.at[pl.ds(s * RPT, RPT)])
            plsc.subcore_barrier()

            def body(j, carry):
                pltpu.sync_copy(u_ref.at[srcv.at[j]], rowbuf)
                pltpu.sync_copy(rowbuf, acc_sh.at[dstv.at[j]], add=True)
                return carry

            lax.fori_loop(0, NB, body, 0)
            plsc.subcore_barrier()
            pltpu.sync_copy(acc_sh.at[pl.ds(s * RPT, RPT)],
                            agg_hbm.at[chunk, pl.ds(s * RPT, RPT)])
            plsc.subcore_barrier()


_agg_wide_call = pl.kernel(
    _agg_wide_body,
    out_type=_sds((4, NPAD, 128)),
    mesh=_mesh,
    scratch_types=[
        pltpu.VMEM((NB, BB), jnp.int32),
        pltpu.VMEM((NB, BB), jnp.int32),
        pltpu.VMEM((BB, 128), jnp.float32),
        pltpu.VMEM_SHARED((NPAD, 128), jnp.float32),
    ],
)


def _agg_narrow_body(u_hbm, src_hbm, dst_hbm, agg_hbm,
                     srcv, dstv, rowbuf, acc_sh):
    c = lax.axis_index("c")
    s = lax.axis_index("s")
    pltpu.sync_copy(src_hbm.at[s], srcv)
    pltpu.sync_copy(dst_hbm.at[s], dstv)
    pltpu.sync_copy(u_hbm.at[pl.ds(s * RPT, RPT)],
                    acc_sh.at[pl.ds(s * RPT, RPT)])
    plsc.subcore_barrier()

    def body(j, carry):
        jb = 2 * j + c

        @pl.when(jb < NB)
        def _():
            pltpu.sync_copy(u_hbm.at[srcv.at[jb]], rowbuf)
            pltpu.sync_copy(rowbuf, acc_sh.at[dstv.at[jb]], add=True)
        return carry

    lax.fori_loop(0, (NB + 1) // 2, body, 0)
    plsc.subcore_barrier()
    pltpu.sync_copy(acc_sh.at[pl.ds(s * RPT, RPT)],
                    agg_hbm.at[c, pl.ds(s * RPT, RPT)])


_agg_narrow_call = pl.kernel(
    _agg_narrow_body,
    out_type=_sds((NC, NPAD, 128)),
    mesh=_mesh,
    scratch_types=[
        pltpu.VMEM((NB, BB), jnp.int32),
        pltpu.VMEM((NB, BB), jnp.int32),
        pltpu.VMEM((BB, 128), jnp.float32),
        pltpu.VMEM_SHARED((NPAD, 128), jnp.float32),
    ],
)


def _dinv_body(deg_ref, out_ref):
    d = deg_ref[...]
    out_ref[...] = lax.rsqrt(d[0] + d[1] + 1.0)


def _dinv_call(deg2):
    return pl.pallas_call(
        _dinv_body,
        out_shape=_sds((NPAD // 128, 128)),
    )(deg2)


def _mm1_body(x_ref, w_ref, dinv_ref, o0, o1, o2, o3):
    h = jnp.dot(x_ref[...], w_ref[...], preferred_element_type=jnp.float32)
    u = h * dinv_ref[...]
    for i, o in enumerate((o0, o1, o2, o3)):
        o[...] = u[:, i * 128:(i + 1) * 128]


def _mm1_call(x, w, dinv):
    return pl.pallas_call(
        _mm1_body,
        grid=(GR,),
        in_specs=[
            pl.BlockSpec((MB, F_IN), lambda i: (i, 0)),
            pl.BlockSpec((F_IN, H), lambda i: (0, 0)),
            pl.BlockSpec((MB, 1), lambda i: (i, 0)),
        ],
        out_specs=[pl.BlockSpec((MB, 128), lambda i: (i, 0))] * 4,
        out_shape=[_sds((NPAD, 128))] * 4,
    )(x, w, dinv)


def _stats_body(a0, a1, a2, a3, dinv_ref, b_ref, ps_ref, pq_ref):
    i = pl.program_id(0)
    y = jnp.concatenate([a0[...], a1[...], a2[...], a3[...]], axis=1)
    y = y * dinv_ref[...] + b_ref[...]

    @pl.when(i == 0)
    def _():
        ps_ref[...] = jnp.zeros_like(ps_ref)
        pq_ref[...] = jnp.zeros_like(pq_ref)

    ps_ref[...] += jnp.sum(y, axis=0, keepdims=True)
    pq_ref[...] += jnp.sum(y * y, axis=0, keepdims=True)


def _stats_call(chunks, dinv, b):
    return pl.pallas_call(
        _stats_body,
        grid=(GR,),
        in_specs=[pl.BlockSpec((MB, 128), lambda i: (i, 0))] * 4
        + [
            pl.BlockSpec((MB, 1), lambda i: (i, 0)),
            pl.BlockSpec((1, H), lambda i: (0, 0)),
        ],
        out_specs=[pl.BlockSpec((1, H), lambda i: (0, 0))] * 2,
        out_shape=[_sds((1, H))] * 2,
    )(*chunks, dinv, b)


def _bnmm_body(n_out, a0, a1, a2, a3, dinv_ref, b_ref, ps_ref, pq_ref,
               g_ref, be_ref, w_ref, *outs):
    mean = ps_ref[...] / N
    var = pq_ref[...] / N - mean * mean
    scale = g_ref[...] * lax.rsqrt(var + EPS)
    shift = be_ref[...] - mean * scale
    y = jnp.concatenate([a0[...], a1[...], a2[...], a3[...]], axis=1)
    y = (y * dinv_ref[...] + b_ref[...]) * scale + shift
    z = jnp.maximum(y, 0.0)
    u = jnp.dot(z, w_ref[...], preferred_element_type=jnp.float32)
    u = u * dinv_ref[...]
    if n_out == 1:
        outs[0][...] = u
    else:
        for i, o in enumerate(outs):
            o[...] = u[:, i * 128:(i + 1) * 128]


def _bnmm_call(chunks, dinv, b, ps, pq, g, be, w, n_out, wout):
    return pl.pallas_call(
        functools.partial(_bnmm_body, n_out),
        grid=(GR,),
        in_specs=[pl.BlockSpec((MB, 128), lambda i: (i, 0))] * 4
        + [
            pl.BlockSpec((MB, 1), lambda i: (i, 0)),
            pl.BlockSpec((1, H), lambda i: (0, 0)),
            pl.BlockSpec((1, H), lambda i: (0, 0)),
            pl.BlockSpec((1, H), lambda i: (0, 0)),
            pl.BlockSpec((1, H), lambda i: (0, 0)),
            pl.BlockSpec((1, H), lambda i: (0, 0)),
            pl.BlockSpec((H, wout * n_out), lambda i: (0, 0)),
        ],
        out_specs=[pl.BlockSpec((MB, wout), lambda i: (i, 0))] * n_out,
        out_shape=[_sds((NPAD, wout))] * n_out,
    )(*chunks, dinv, b, ps, pq, g, be, w)


def _final_body(s0, s1, u_ref, dinv_ref, b_ref, out_ref):
    r = (s0[...] + s1[...] - u_ref[...]) * dinv_ref[...]
    out_ref[...] = r[:, :C] + b_ref[...]


def _final_call(s0, s1, u, dinv, b):
    return pl.pallas_call(
        _final_body,
        grid=(GR,),
        in_specs=[
            pl.BlockSpec((MB, 128), lambda i: (i, 0)),
            pl.BlockSpec((MB, 128), lambda i: (i, 0)),
            pl.BlockSpec((MB, 128), lambda i: (i, 0)),
            pl.BlockSpec((MB, 1), lambda i: (i, 0)),
            pl.BlockSpec((1, C), lambda i: (0, 0)),
        ],
        out_specs=pl.BlockSpec((MB, C), lambda i: (i, 0)),
        out_shape=_sds((N, C)),
    )(s0, s1, u, dinv, b)


def kernel(x, edge_idx, W1, b1, g1, be1, W2, b2, g2, be2, W3, b3):
    src = edge_idx[0].reshape(NS, ET)
    dst = edge_idx[1].reshape(NS, ET)
    srcp = jnp.pad(src, ((0, 0), (0, ETP - ET))).reshape(NS, NB, BB)
    dstp = jnp.pad(dst, ((0, 0), (0, ETP - ET)),
                   constant_values=N).reshape(NS, NB, BB)
    ones = jnp.ones((BB, 128), jnp.float32)
    zeros = jnp.zeros((NPAD, 128), jnp.float32)

    deg = _deg_call(dstp, ones, zeros)
    deg2 = deg[:, :, 0].reshape(NC, NPAD // 128, 128)
    dinv = _dinv_call(deg2).reshape(NPAD)[:N].reshape(N, 1)

    b1r, g1r, be1r = b1.reshape(1, H), g1.reshape(1, H), be1.reshape(1, H)
    b2r, g2r, be2r = b2.reshape(1, H), g2.reshape(1, H), be2.reshape(1, H)
    b3r = b3.reshape(1, C)

    u1 = _mm1_call(x, W1, dinv)
    agg1 = _agg_wide_call(*u1, srcp, dstp)
    a1 = [agg1[i, :N] for i in range(4)]
    ps1, pq1 = _stats_call(a1, dinv, b1r)
    u2 = _bnmm_call(a1, dinv, b1r, ps1, pq1, g1r, be1r, W2, 4, 128)
    agg2 = _agg_wide_call(*u2, srcp, dstp)
    a2 = [agg2[i, :N] for i in range(4)]
    ps2, pq2 = _stats_call(a2, dinv, b2r)
    W3p = jnp.pad(W3, ((0, 0), (0, 128 - C)))
    (u3,) = _bnmm_call(a2, dinv, b2r, ps2, pq2, g2r, be2r, W3p, 1, 128)
    agg3 = _agg_narrow_call(u3, srcp, dstp)
    out = _final_call(agg3[0, :N], agg3[1, :N], u3[:N], dinv, b3r)
    return out
